# Initial kernel scaffold; baseline (speedup 1.0000x reference)
import jax
import jax.numpy as jnp
from jax.experimental import pallas as pl


def kernel(x, Wr, br, Wn, bn, We, be):
    raise NotImplementedError("write your pallas kernel here")


# speedup vs baseline: 1.6152x; 1.6152x over previous
import jax
import jax.numpy as jnp
from jax.experimental import pallas as pl

E = 8
D = 512
T = 8192
TB = 512
LANES = 128
DP = D + 8


def _moe_body(xa_ref, wrn_ref, eps_ref, wet_ref, be_ref, o_ref):
    xa = xa_ref[...]
    xb = xa[:, :D]
    logits_all = jnp.dot(xa, wrn_ref[...],
                         preferred_element_type=jnp.float32)
    logits = logits_all[:, :E]
    noise = logits_all[:, E:2 * E]
    eps = eps_ref[...][:, :E]
    noisy = logits + eps * jax.nn.softplus(noise)

    lane = jax.lax.broadcasted_iota(jnp.int32, (TB, E), 1)
    NEG = jnp.float32(-1e30)
    m1 = jnp.max(noisy, axis=1, keepdims=True)
    i1 = jnp.min(jnp.where(noisy == m1, lane, E), axis=1, keepdims=True)
    n2 = jnp.where(lane == i1, NEG, noisy)
    m2 = jnp.max(n2, axis=1, keepdims=True)
    i2 = jnp.min(jnp.where(n2 == m2, lane, E), axis=1, keepdims=True)
    e2 = jnp.exp(m2 - m1)
    w1 = 1.0 / (1.0 + e2)
    w2 = e2 / (1.0 + e2)

    acc = jnp.zeros((TB, D), jnp.float32)
    for i in range(E):
        g = jnp.where(i1 == i, w1, 0.0) + jnp.where(i2 == i, w2, 0.0)
        y = jnp.dot(xb, wet_ref[i], preferred_element_type=jnp.float32)
        acc = acc + g * (y + be_ref[i][None, :])
    o_ref[...] = acc


def kernel(x, Wr, br, Wn, bn, We, be):
    eps = jax.random.normal(jax.random.key(42), (T, E), dtype=jnp.float32)
    eps_pad = jnp.zeros((T, LANES), jnp.float32).at[:, :E].set(eps)
    xa = jnp.concatenate(
        [x, jnp.ones((T, 1), jnp.float32), jnp.zeros((T, DP - D - 1), jnp.float32)],
        axis=1)
    wrn_a = jnp.zeros((DP, LANES), jnp.float32)
    wrn_a = wrn_a.at[:D, :E].set(Wr).at[:D, E:2 * E].set(Wn)
    wrn_a = wrn_a.at[D, :E].set(br).at[D, E:2 * E].set(bn)
    wet = jnp.swapaxes(We, 1, 2)

    out = pl.pallas_call(
        _moe_body,
        grid=(T // TB,),
        in_specs=[
            pl.BlockSpec((TB, DP), lambda b: (b, 0)),
            pl.BlockSpec((DP, LANES), lambda b: (0, 0)),
            pl.BlockSpec((TB, LANES), lambda b: (b, 0)),
            pl.BlockSpec((E, D, D), lambda b: (0, 0, 0)),
            pl.BlockSpec((E, D), lambda b: (0, 0)),
        ],
        out_specs=pl.BlockSpec((TB, D), lambda b: (b, 0)),
        out_shape=jax.ShapeDtypeStruct((T, D), jnp.float32),
    )(xa, wrn_a, eps_pad, wet, be)
    return out


# speedup vs baseline: 1.6206x; 1.0033x over previous
import jax
import jax.numpy as jnp
from jax.experimental import pallas as pl

E = 8
D = 512
T = 8192
TB = 512
LANES = 128
DP = D + 8


def _moe_body(xa_ref, wrn_ref, eps_ref, wet_ref, be_ref, o_ref):
    xa = xa_ref[...]
    xb = xa[:, :D]
    logits_all = jnp.dot(xa, wrn_ref[...],
                         preferred_element_type=jnp.float32)
    logits = logits_all[:, :E]
    noise = logits_all[:, E:2 * E]
    eps = eps_ref[...][:, :E]
    noisy = logits + eps * jax.nn.softplus(noise)

    lane = jax.lax.broadcasted_iota(jnp.int32, (TB, E), 1)
    NEG = jnp.float32(-1e30)
    m1 = jnp.max(noisy, axis=1, keepdims=True)
    i1 = jnp.min(jnp.where(noisy == m1, lane, E), axis=1, keepdims=True)
    n2 = jnp.where(lane == i1, NEG, noisy)
    m2 = jnp.max(n2, axis=1, keepdims=True)
    i2 = jnp.min(jnp.where(n2 == m2, lane, E), axis=1, keepdims=True)
    e2 = jnp.exp(m2 - m1)
    w1 = 1.0 / (1.0 + e2)
    w2 = e2 / (1.0 + e2)

    acc = jnp.zeros((TB, D), jnp.float32)
    xb16 = xb.astype(jnp.bfloat16)
    for i in range(E):
        g = jnp.where(i1 == i, w1, 0.0) + jnp.where(i2 == i, w2, 0.0)
        y = jnp.dot(xb16, wet_ref[i].astype(jnp.bfloat16),
                    preferred_element_type=jnp.float32)
        acc = acc + g * (y + be_ref[i][None, :])
    o_ref[...] = acc


def kernel(x, Wr, br, Wn, bn, We, be):
    eps = jax.random.normal(jax.random.key(42), (T, E), dtype=jnp.float32)
    eps_pad = jnp.zeros((T, LANES), jnp.float32).at[:, :E].set(eps)
    xa = jnp.concatenate(
        [x, jnp.ones((T, 1), jnp.float32), jnp.zeros((T, DP - D - 1), jnp.float32)],
        axis=1)
    wrn_a = jnp.zeros((DP, LANES), jnp.float32)
    wrn_a = wrn_a.at[:D, :E].set(Wr).at[:D, E:2 * E].set(Wn)
    wrn_a = wrn_a.at[D, :E].set(br).at[D, E:2 * E].set(bn)
    wet = jnp.swapaxes(We, 1, 2)

    out = pl.pallas_call(
        _moe_body,
        grid=(T // TB,),
        in_specs=[
            pl.BlockSpec((TB, DP), lambda b: (b, 0)),
            pl.BlockSpec((DP, LANES), lambda b: (0, 0)),
            pl.BlockSpec((TB, LANES), lambda b: (b, 0)),
            pl.BlockSpec((E, D, D), lambda b: (0, 0, 0)),
            pl.BlockSpec((E, D), lambda b: (0, 0)),
        ],
        out_specs=pl.BlockSpec((TB, D), lambda b: (b, 0)),
        out_shape=jax.ShapeDtypeStruct((T, D), jnp.float32),
    )(xa, wrn_a, eps_pad, wet, be)
    return out
